# baseline (speedup 1.0000x reference)
import functools

import jax
import jax.numpy as jnp
from jax import lax
from jax.experimental import pallas as pl
from jax.experimental.pallas import tpu as pltpu
from jax.experimental.pallas import tpu_sc as plsc

_N = 65536
_D = 32
_K = 512
_BLK = 2048

_NC = 2
_NS = 16
_NW = _NC * _NS
_BPW = _N // _NW
_CHUNK = 128
_NCHUNK = _BPW // _CHUNK
_AHEAD = 4


def _tc_body(x_ref, v_ref, idx_ref, idx2_ref, vt_ref, loss_ref):
    xb = x_ref[...]
    v = v_ref[...]
    xv = jnp.dot(xb, v, preferred_element_type=jnp.float32)
    rownorm = jnp.sum(xb * xb, axis=1, keepdims=True)
    vnorm = jnp.sum(v * v, axis=0, keepdims=True)
    d = (rownorm - 2.0 * xv) + vnorm
    m = jnp.min(d, axis=1, keepdims=True)
    iota = lax.broadcasted_iota(jnp.int32, (1, _K), 1)
    idx = jnp.min(jnp.where(d == m, iota, _K), axis=1)
    idx_ref[...] = idx[:, None]
    idx2_ref[...] = idx.reshape(_BLK // _CHUNK, _CHUNK)

    @pl.when(pl.program_id(0) == 0)
    def _():
        loss_ref[0] = 0.0
        vt_ref[...] = jnp.concatenate(
            [v.T, jnp.zeros((_K, 128 - _D), jnp.float32)], axis=1)

    loss_ref[0] += jnp.sum(m)


def _tc_part(x, vectors):
    grid = _N // _BLK
    return pl.pallas_call(
        _tc_body,
        grid=(grid,),
        in_specs=[
            pl.BlockSpec((_BLK, _D), lambda i: (i, 0)),
            pl.BlockSpec((_D, _K), lambda i: (0, 0)),
        ],
        out_specs=[
            pl.BlockSpec((_BLK, 1), lambda i: (i, 0)),
            pl.BlockSpec((_BLK // _CHUNK, _CHUNK), lambda i: (i, 0)),
            pl.BlockSpec((_K, 128), lambda i: (0, 0)),
            pl.BlockSpec(memory_space=pltpu.SMEM),
        ],
        out_shape=[
            jax.ShapeDtypeStruct((_N, 1), jnp.int32),
            jax.ShapeDtypeStruct((_N // _CHUNK, _CHUNK), jnp.int32),
            jax.ShapeDtypeStruct((_K, 128), jnp.float32),
            jax.ShapeDtypeStruct((1,), jnp.float32),
        ],
    )(x, vectors)


@functools.partial(
    pl.kernel,
    out_type=jax.ShapeDtypeStruct((_N, _D), jnp.float32),
    mesh=plsc.VectorSubcoreMesh(core_axis_name="c", subcore_axis_name="s"),
    scratch_types=[
        pltpu.VMEM((_NCHUNK, _CHUNK), jnp.int32),
        pltpu.VMEM((_AHEAD, _CHUNK, 128), jnp.float32),
        pltpu.VMEM((2, _CHUNK, _D), jnp.float32),
        pltpu.SemaphoreType.DMA,
        pltpu.SemaphoreType.DMA,
    ],
)
def _sc_gather(table_hbm, idx_hbm, out_hbm, idx_v, rows_v, outc_v, gsem, osem):
    wid = lax.axis_index("s") * _NC + lax.axis_index("c")
    base = wid * _BPW
    pltpu.sync_copy(idx_hbm.at[pl.ds(wid * _NCHUNK, _NCHUNK)], idx_v)
    for j in range(_AHEAD):
        pltpu.async_copy(table_hbm.at[idx_v.at[j]], rows_v.at[j], gsem)
    for j in range(_NCHUNK):
        pltpu.make_async_copy(table_hbm.at[idx_v.at[j]],
                              rows_v.at[j % _AHEAD], gsem).wait()
        b = j % _AHEAD
        o = j % 2
        if j >= 2:
            pltpu.make_async_copy(
                outc_v.at[o],
                out_hbm.at[pl.ds(base + (j - 2) * _CHUNK, _CHUNK)],
                osem).wait()
        for p in range(_CHUNK):
            outc_v[o, p, pl.ds(0, 16)] = rows_v[b, p, pl.ds(0, 16)]
            outc_v[o, p, pl.ds(16, 16)] = rows_v[b, p, pl.ds(16, 16)]
        if j + _AHEAD < _NCHUNK:
            pltpu.async_copy(table_hbm.at[idx_v.at[j + _AHEAD]],
                             rows_v.at[(j + _AHEAD) % _AHEAD], gsem)
        pltpu.async_copy(outc_v.at[o],
                         out_hbm.at[pl.ds(base + j * _CHUNK, _CHUNK)], osem)
    for j in range(_NCHUNK - 2, _NCHUNK):
        pltpu.make_async_copy(
            outc_v.at[j % 2],
            out_hbm.at[pl.ds(base + j * _CHUNK, _CHUNK)], osem).wait()


def kernel(x, vectors):
    idx, idx2, vt, loss_sum = _tc_part(x, vectors)
    q = _sc_gather(vt, idx2)
    loss = loss_sum[0] / (_N * _D)
    return (q, loss, loss, idx)


# speedup vs baseline: 1.1169x; 1.1169x over previous
import functools

import jax
import jax.numpy as jnp
from jax import lax
from jax.experimental import pallas as pl
from jax.experimental.pallas import tpu as pltpu
from jax.experimental.pallas import tpu_sc as plsc

_N = 65536
_D = 32
_K = 512
_BLK = 2048
_CHUNK = 128

_NC = 2
_NS = 16
_NW = _NC * _NS

_NSPLIT = 2
_NH = _N // _NSPLIT


def _tc_body(x_ref, v_ref, idx_ref, idx2_ref, vt_ref, loss_ref):
    xb = x_ref[...]
    v = v_ref[...]
    xv = jnp.dot(xb, v, preferred_element_type=jnp.float32)
    rownorm = jnp.sum(xb * xb, axis=1, keepdims=True)
    vnorm = jnp.sum(v * v, axis=0, keepdims=True)
    d = (rownorm - 2.0 * xv) + vnorm
    m = jnp.min(d, axis=1, keepdims=True)
    iota = lax.broadcasted_iota(jnp.int32, (1, _K), 1)
    idx = jnp.min(jnp.where(d == m, iota, _K), axis=1)
    idx_ref[...] = idx[:, None]
    idx2_ref[...] = idx.reshape(_BLK // _CHUNK, _CHUNK)

    @pl.when(pl.program_id(0) == 0)
    def _():
        loss_ref[0] = 0.0
        vt_ref[...] = v.T

    loss_ref[0] += jnp.sum(m)


def _tc_part(x, vectors):
    n = x.shape[0]
    grid = n // _BLK
    return pl.pallas_call(
        _tc_body,
        grid=(grid,),
        in_specs=[
            pl.BlockSpec((_BLK, _D), lambda i: (i, 0)),
            pl.BlockSpec((_D, _K), lambda i: (0, 0)),
        ],
        out_specs=[
            pl.BlockSpec((_BLK, 1), lambda i: (i, 0)),
            pl.BlockSpec((_BLK // _CHUNK, _CHUNK), lambda i: (i, 0)),
            pl.BlockSpec((_K, _D), lambda i: (0, 0)),
            pl.BlockSpec(memory_space=pltpu.SMEM),
        ],
        out_shape=[
            jax.ShapeDtypeStruct((n, 1), jnp.int32),
            jax.ShapeDtypeStruct((n // _CHUNK, _CHUNK), jnp.int32),
            jax.ShapeDtypeStruct((_K, _D), jnp.float32),
            jax.ShapeDtypeStruct((1,), jnp.float32),
        ],
    )(x, vectors)


def _make_sc_gather(n):
    bpw = n // _NW
    nrow = bpw // _CHUNK
    unroll = 8

    @functools.partial(
        pl.kernel,
        out_type=jax.ShapeDtypeStruct((n, _D), jnp.float32),
        mesh=plsc.VectorSubcoreMesh(core_axis_name="c", subcore_axis_name="s"),
        scratch_types=[
            pltpu.VMEM((_K, _D), jnp.float32),
            pltpu.VMEM((nrow, _CHUNK), jnp.int32),
            pltpu.VMEM((bpw, _D), jnp.float32),
        ],
        compiler_params=pltpu.CompilerParams(
            needs_layout_passes=False, use_tc_tiling_on_sc=False),
    )
    def sc_gather(table_hbm, idx_hbm, out_hbm, table_v, idx_v, out_v):
        wid = lax.axis_index("s") * _NC + lax.axis_index("c")
        base = wid * bpw
        pltpu.sync_copy(table_hbm, table_v)
        pltpu.sync_copy(idx_hbm.at[pl.ds(wid * nrow, nrow)], idx_v)
        lane = lax.broadcasted_iota(jnp.int32, (16,), 0)
        lane_hi = lane + 16

        def row_body(c):
            def body(i, _):
                for u in range(unroll):
                    p = i * unroll + u
                    kv = plsc.load_gather(
                        idx_v, [jnp.full((16,), c, jnp.int32),
                                jnp.broadcast_to(p, (16,)).astype(jnp.int32)])
                    lo = plsc.load_gather(table_v, [kv, lane])
                    hi = plsc.load_gather(table_v, [kv, lane_hi])
                    q = c * _CHUNK + p
                    out_v[q, pl.ds(0, 16)] = lo
                    out_v[q, pl.ds(16, 16)] = hi
                return 0

            lax.fori_loop(0, _CHUNK // unroll, body, 0)

        for c in range(nrow):
            row_body(c)
        pltpu.sync_copy(out_v, out_hbm.at[pl.ds(base, bpw)])

    return sc_gather


_sc_gather_h = _make_sc_gather(_NH)


def kernel(x, vectors):
    idxs, qs, loss_sum = [], [], 0.0
    for h in range(_NSPLIT):
        xh = lax.slice_in_dim(x, h * _NH, (h + 1) * _NH, axis=0)
        idx, idx2, vt, ls = _tc_part(xh, vectors)
        idxs.append(idx)
        qs.append(_sc_gather_h(vt, idx2))
        loss_sum = loss_sum + ls[0]
    q = jnp.concatenate(qs, axis=0)
    idx = jnp.concatenate(idxs, axis=0)
    loss = loss_sum / (_N * _D)
    return (q, loss, loss, idx)


# speedup vs baseline: 1.1806x; 1.0570x over previous
import functools

import jax
import jax.numpy as jnp
from jax import lax
from jax.experimental import pallas as pl
from jax.experimental.pallas import tpu as pltpu
from jax.experimental.pallas import tpu_sc as plsc

_N = 65536
_D = 32
_K = 512
_BLK = 2048
_CHUNK = 128

_NC = 2
_NS = 16
_NW = _NC * _NS

_NSPLIT = 2
_NH = _N // _NSPLIT


def _tc_body(x_ref, v_ref, idx_ref, idx2_ref, vt_ref, loss_ref):
    xb = x_ref[...]
    v = v_ref[...]
    xv = jnp.dot(xb, v, preferred_element_type=jnp.float32)
    rownorm = jnp.sum(xb * xb, axis=1, keepdims=True)
    vnorm = jnp.sum(v * v, axis=0, keepdims=True)
    d = (rownorm - 2.0 * xv) + vnorm
    m = jnp.min(d, axis=1, keepdims=True)
    iota = lax.broadcasted_iota(jnp.int32, (1, _K), 1)
    idx = jnp.min(jnp.where(d == m, iota, _K), axis=1)
    idx_ref[...] = idx[:, None]
    idx2_ref[...] = idx.reshape(_BLK // _CHUNK, _CHUNK)

    @pl.when(pl.program_id(0) == 0)
    def _():
        loss_ref[0] = 0.0
        vt_ref[...] = v.T

    loss_ref[0] += jnp.sum(m)


def _tc_part(x, vectors):
    n = x.shape[0]
    grid = n // _BLK
    return pl.pallas_call(
        _tc_body,
        grid=(grid,),
        in_specs=[
            pl.BlockSpec((_BLK, _D), lambda i: (i, 0)),
            pl.BlockSpec((_D, _K), lambda i: (0, 0)),
        ],
        out_specs=[
            pl.BlockSpec((_BLK, 1), lambda i: (i, 0)),
            pl.BlockSpec((_BLK // _CHUNK, _CHUNK), lambda i: (i, 0)),
            pl.BlockSpec((_K, _D), lambda i: (0, 0)),
            pl.BlockSpec(memory_space=pltpu.SMEM),
        ],
        out_shape=[
            jax.ShapeDtypeStruct((n, 1), jnp.int32),
            jax.ShapeDtypeStruct((n // _CHUNK, _CHUNK), jnp.int32),
            jax.ShapeDtypeStruct((_K, _D), jnp.float32),
            jax.ShapeDtypeStruct((1,), jnp.float32),
        ],
    )(x, vectors)


def _make_sc_gather(n):
    bpw = n // _NW
    nrow = bpw // _CHUNK
    unroll = 8

    @functools.partial(
        pl.kernel,
        out_type=jax.ShapeDtypeStruct((n, _D), jnp.float32),
        mesh=plsc.VectorSubcoreMesh(core_axis_name="c", subcore_axis_name="s"),
        scratch_types=[
            pltpu.VMEM((_K, _D), jnp.float32),
            pltpu.VMEM((nrow, _CHUNK), jnp.int32),
            pltpu.VMEM((bpw, _D), jnp.float32),
        ],
        compiler_params=pltpu.CompilerParams(
            needs_layout_passes=False, use_tc_tiling_on_sc=False),
    )
    def sc_gather(table_hbm, idx_hbm, out_hbm, table_v, idx_v, out_v):
        wid = lax.axis_index("s") * _NC + lax.axis_index("c")
        base = wid * bpw
        pltpu.sync_copy(table_hbm, table_v)
        pltpu.sync_copy(idx_hbm.at[pl.ds(wid * nrow, nrow)], idx_v)
        lane = lax.broadcasted_iota(jnp.int32, (16,), 0)
        lane_hi = lane + 16

        def row_body(c):
            def body(i, _):
                for u in range(unroll):
                    p = i * unroll + u
                    kv = plsc.load_gather(
                        idx_v, [jnp.full((16,), c, jnp.int32),
                                jnp.broadcast_to(p, (16,)).astype(jnp.int32)])
                    lo = plsc.load_gather(table_v, [kv, lane])
                    hi = plsc.load_gather(table_v, [kv, lane_hi])
                    q = c * _CHUNK + p
                    out_v[q, pl.ds(0, 16)] = lo
                    out_v[q, pl.ds(16, 16)] = hi
                return 0

            lax.fori_loop(0, _CHUNK // unroll, body, 0)

        for c in range(nrow):
            row_body(c)
        pltpu.sync_copy(out_v, out_hbm.at[pl.ds(base, bpw)])

    return sc_gather


_sc_gather_full = _make_sc_gather(_N)


def kernel(x, vectors):
    idx, idx2, vt, ls = _tc_part(x, vectors)
    q = _sc_gather_full(vt, idx2)
    loss = ls[0] / (_N * _D)
    return (q, loss, loss, idx)
